# baseline (speedup 1.0000x reference)
import jax
import jax.numpy as jnp
from jax.experimental import pallas as pl

N = 1024
P = 512
C_IN = 64
C_OUT = 128
KS = 24
STRIDE = 2
SIGMA = 0.2
NN = 32
NA = 12

AK = NA * KS
CK = C_IN * KS
PA = P * NA


def _knn_body(xq_ref, xs_ref, idx_ref):
    d2 = jnp.zeros((P, N), jnp.float32)
    for d in range(3):
        diff = xq_ref[:, d : d + 1] - xs_ref[d : d + 1, :]
        d2 = d2 + diff * diff
    lane = jax.lax.broadcasted_iota(jnp.int32, (P, N), 1)
    cols = []
    for _ in range(NN):
        m = jnp.min(d2, axis=1, keepdims=True)
        cand = jnp.where(d2 == m, lane, N)
        idx_t = jnp.min(cand, axis=1, keepdims=True)
        cols.append(idx_t)
        d2 = jnp.where(cand == idx_t, jnp.inf, d2)
    idx_ref[...] = jnp.concatenate(cols, axis=1)


def _fused_body(pk_ref, xs_ref, rk_ref, f2_ref, wmat_ref, w_ref, out_ref):
    rows = pk_ref.shape[0]
    pb = rows // NN
    lane = jax.lax.broadcasted_iota(jnp.int32, (rows, N), 1).astype(jnp.float32)
    sel = lane == pk_ref[:, 0:1]
    onehot = sel.astype(jnp.float32)
    gf = jax.lax.dot_general(
        onehot, f2_ref[...], (((1,), (0,)), ((), ())),
        preferred_element_type=jnp.float32,
    )

    acc = jnp.zeros((rows, AK), jnp.float32)
    for d in range(3):
        g = jnp.sum(jnp.where(sel, xs_ref[d : d + 1, :], 0.0), axis=1, keepdims=True)
        rel = g - pk_ref[:, 1 + d : 2 + d]
        diff = rel - rk_ref[d : d + 1, :]
        acc = acc + diff * diff
    dist = jnp.sqrt(acc + 1e-12)
    wgt = jnp.maximum(1.0 - dist / SIGMA, 0.0)
    w_ref[...] = wgt
    AS = 4
    convs = []
    for g in range(NA // AS):
        gf3 = gf[:, g * AS * C_IN : (g + 1) * AS * C_IN].reshape(pb, NN, AS * C_IN)
        w3 = wgt[:, g * AS * KS : (g + 1) * AS * KS].reshape(pb, NN, AS * KS)
        tmp = jax.lax.dot_general(
            gf3, w3, (((1,), (1,)), ((0,), (0,))),
            preferred_element_type=jnp.float32,
        )
        for j in range(AS):
            conv_a = tmp[:, j * C_IN : (j + 1) * C_IN, j * KS : (j + 1) * KS]
            convs.append(conv_a.reshape(pb, CK))
    conv_all = jnp.concatenate(convs, axis=0)
    out_flat = jax.lax.dot_general(
        conv_all, wmat_ref[...], (((1,), (1,)), ((), ())),
        preferred_element_type=jnp.float32,
    )
    out_ref[...] = out_flat.reshape(NA, pb, C_OUT)


def _norm_body(x_ref, o_ref):
    x = x_ref[...]
    mu = jnp.sum(jnp.sum(x, axis=1, keepdims=True), axis=0, keepdims=True) / PA
    xc = x - mu
    var = jnp.sum(jnp.sum(xc * xc, axis=1, keepdims=True), axis=0, keepdims=True) / PA
    o_ref[...] = jnp.maximum(xc / jnp.sqrt(var + 1e-5), 0.0)


def kernel(xyz, feats, anchors, W, kernels):
    xs = xyz[0]
    xq = jnp.transpose(xs, (1, 0))[::STRIDE]

    idx = pl.pallas_call(
        _knn_body,
        out_shape=jax.ShapeDtypeStruct((P, NN), jnp.int32),
    )(xq, xs)

    idxf = idx.astype(jnp.float32).reshape(P * NN, 1)
    nxr = jnp.repeat(xq, NN, axis=0)
    pk = jnp.concatenate([idxf, nxr], axis=1)

    rk = jnp.einsum("aij,kj->aki", anchors, kernels)
    rk2 = jnp.transpose(rk.reshape(AK, 3), (1, 0))

    f2 = jnp.transpose(feats[0], (1, 2, 0)).reshape(N, NA * C_IN)

    n_cblk = 8
    pb = P // n_cblk
    crows = pb * NN
    w, out_raw = pl.pallas_call(
        _fused_body,
        grid=(n_cblk,),
        in_specs=[
            pl.BlockSpec((crows, 4), lambda i: (i, 0)),
            pl.BlockSpec((3, N), lambda i: (0, 0)),
            pl.BlockSpec((3, AK), lambda i: (0, 0)),
            pl.BlockSpec((N, NA * C_IN), lambda i: (0, 0)),
            pl.BlockSpec((C_OUT, CK), lambda i: (0, 0)),
        ],
        out_specs=[
            pl.BlockSpec((crows, AK), lambda i: (i, 0)),
            pl.BlockSpec((NA, pb, C_OUT), lambda i: (0, i, 0)),
        ],
        out_shape=[
            jax.ShapeDtypeStruct((P * NN, AK), jnp.float32),
            jax.ShapeDtypeStruct((NA, P, C_OUT), jnp.float32),
        ],
    )(pk, xs, rk2, f2, W)

    feat = pl.pallas_call(
        _norm_body,
        out_shape=jax.ShapeDtypeStruct((NA, P, C_OUT), jnp.float32),
    )(out_raw)

    inter_idx = idx.reshape(1, P, NN)
    inter_w = w.reshape(1, P, NN, NA, KS)
    sample_idx = jnp.arange(0, N, STRIDE)
    new_xyz = xyz[:, :, ::STRIDE]
    feat_out = jnp.transpose(feat, (2, 1, 0)).reshape(1, C_OUT, P, NA)
    return inter_idx, inter_w, sample_idx, new_xyz, feat_out, anchors


# speedup vs baseline: 1.2162x; 1.2162x over previous
import jax
import jax.numpy as jnp
from jax.experimental import pallas as pl

N = 1024
P = 512
C_IN = 64
C_OUT = 128
KS = 24
STRIDE = 2
SIGMA = 0.2
NN = 32
NA = 12

AK = NA * KS
CK = C_IN * KS
PA = P * NA


def _knn_body(xq_ref, xs_ref, idx_ref):
    d2 = jnp.zeros((P, N), jnp.float32)
    for d in range(3):
        diff = xq_ref[:, d : d + 1] - xs_ref[d : d + 1, :]
        d2 = d2 + diff * diff
    lane = jax.lax.broadcasted_iota(jnp.int32, (P, N), 1)
    cols = []
    for _ in range(NN):
        m = jnp.min(d2, axis=1, keepdims=True)
        cand = jnp.where(d2 == m, lane, N)
        idx_t = jnp.min(cand, axis=1, keepdims=True)
        cols.append(idx_t)
        d2 = jnp.where(cand == idx_t, jnp.inf, d2)
    idx_ref[...] = jnp.concatenate(cols, axis=1)


def _fused_body(pk_ref, xs_ref, rk_ref, f2_ref, wmat_ref, w_ref, out_ref):
    rows = pk_ref.shape[0]
    pb = rows // NN
    lane = jax.lax.broadcasted_iota(jnp.int32, (rows, N), 1).astype(jnp.float32)
    sel = lane == pk_ref[:, 0:1]
    onehot = sel.astype(jnp.bfloat16)
    gf = jax.lax.dot_general(
        onehot, f2_ref[...], (((1,), (0,)), ((), ())),
        preferred_element_type=jnp.float32,
    ).astype(jnp.bfloat16)

    acc = jnp.zeros((rows, AK), jnp.float32)
    for d in range(3):
        g = jnp.sum(jnp.where(sel, xs_ref[d : d + 1, :], 0.0), axis=1, keepdims=True)
        rel = g - pk_ref[:, 1 + d : 2 + d]
        diff = rel - rk_ref[d : d + 1, :]
        acc = acc + diff * diff
    dist = jnp.sqrt(acc + 1e-12)
    wgt = jnp.maximum(1.0 - dist / SIGMA, 0.0)
    w_ref[...] = wgt
    AS = 4
    wgtb = wgt.astype(jnp.bfloat16)
    convs = []
    for g in range(NA // AS):
        gf3 = gf[:, g * AS * C_IN : (g + 1) * AS * C_IN].reshape(pb, NN, AS * C_IN)
        w3 = wgtb[:, g * AS * KS : (g + 1) * AS * KS].reshape(pb, NN, AS * KS)
        tmp = jax.lax.dot_general(
            gf3, w3, (((1,), (1,)), ((0,), (0,))),
            preferred_element_type=jnp.float32,
        ).astype(jnp.bfloat16)
        for j in range(AS):
            conv_a = tmp[:, j * C_IN : (j + 1) * C_IN, j * KS : (j + 1) * KS]
            convs.append(conv_a.reshape(pb, CK))
    conv_all = jnp.concatenate(convs, axis=0)
    out_flat = jax.lax.dot_general(
        conv_all, wmat_ref[...], (((1,), (1,)), ((), ())),
        preferred_element_type=jnp.float32,
    )
    out_ref[...] = out_flat.reshape(NA, pb, C_OUT)


def _norm_body(x_ref, o_ref):
    x = x_ref[...]
    mu = jnp.sum(jnp.sum(x, axis=1, keepdims=True), axis=0, keepdims=True) / PA
    xc = x - mu
    var = jnp.sum(jnp.sum(xc * xc, axis=1, keepdims=True), axis=0, keepdims=True) / PA
    o_ref[...] = jnp.maximum(xc / jnp.sqrt(var + 1e-5), 0.0)


def kernel(xyz, feats, anchors, W, kernels):
    xs = xyz[0]
    xq = jnp.transpose(xs, (1, 0))[::STRIDE]

    idx = pl.pallas_call(
        _knn_body,
        out_shape=jax.ShapeDtypeStruct((P, NN), jnp.int32),
    )(xq, xs)

    idxf = idx.astype(jnp.float32).reshape(P * NN, 1)
    nxr = jnp.repeat(xq, NN, axis=0)
    pk = jnp.concatenate([idxf, nxr], axis=1)

    rk = jnp.einsum("aij,kj->aki", anchors, kernels)
    rk2 = jnp.transpose(rk.reshape(AK, 3), (1, 0))

    f2 = jnp.transpose(feats[0], (1, 2, 0)).reshape(N, NA * C_IN).astype(jnp.bfloat16)
    Wb = W.astype(jnp.bfloat16)

    n_cblk = 8
    pb = P // n_cblk
    crows = pb * NN
    w, out_raw = pl.pallas_call(
        _fused_body,
        grid=(n_cblk,),
        in_specs=[
            pl.BlockSpec((crows, 4), lambda i: (i, 0)),
            pl.BlockSpec((3, N), lambda i: (0, 0)),
            pl.BlockSpec((3, AK), lambda i: (0, 0)),
            pl.BlockSpec((N, NA * C_IN), lambda i: (0, 0)),
            pl.BlockSpec((C_OUT, CK), lambda i: (0, 0)),
        ],
        out_specs=[
            pl.BlockSpec((crows, AK), lambda i: (i, 0)),
            pl.BlockSpec((NA, pb, C_OUT), lambda i: (0, i, 0)),
        ],
        out_shape=[
            jax.ShapeDtypeStruct((P * NN, AK), jnp.float32),
            jax.ShapeDtypeStruct((NA, P, C_OUT), jnp.float32),
        ],
    )(pk, xs, rk2, f2, Wb)

    feat = pl.pallas_call(
        _norm_body,
        out_shape=jax.ShapeDtypeStruct((NA, P, C_OUT), jnp.float32),
    )(out_raw)

    inter_idx = idx.reshape(1, P, NN)
    inter_w = w.reshape(1, P, NN, NA, KS)
    sample_idx = jnp.arange(0, N, STRIDE)
    new_xyz = xyz[:, :, ::STRIDE]
    feat_out = jnp.transpose(feat, (2, 1, 0)).reshape(1, C_OUT, P, NA)
    return inter_idx, inter_w, sample_idx, new_xyz, feat_out, anchors
